# Initial kernel scaffold; baseline (speedup 1.0000x reference)
import functools

import jax
import jax.numpy as jnp
from jax import lax
from jax.experimental import pallas as pl
from jax.experimental.pallas import tpu as pltpu
from jax.experimental.pallas import tpu_sc as plsc

_N = 10000
_E = 320000
_D = 128
_NC = 2
_NS = 16
_NW = _NC * _NS
_EPW = _E // _NW
_K = 80
_NCH = _EPW // _K
_NP = 10240
_RPT = _N // _NS
_ZR = 25

_sc_mesh = plsc.VectorSubcoreMesh(core_axis_name="c", subcore_axis_name="s")


@functools.partial(
    pl.kernel,
    out_type=jax.ShapeDtypeStruct((_NW, _NP), jnp.float32),
    mesh=_sc_mesh,
    scratch_types=[
        pltpu.VMEM((_EPW,), jnp.int32),
        pltpu.VMEM((_NP,), jnp.float32),
    ],
)
def _deg_kernel(di_hbm, out_hbm, didx, hist):
    cid = lax.axis_index("c")
    sid = lax.axis_index("s")
    wid = sid * _NC + cid

    def zbody(i, carry):
        hist[pl.ds(i * 16, 16)] = jnp.zeros((16,), jnp.float32)
        return carry

    lax.fori_loop(0, _NP // 16, zbody, 0)

    pltpu.sync_copy(di_hbm.at[pl.ds(wid * _EPW, _EPW)], didx)
    ones = jnp.ones((16,), jnp.float32)

    def body(i, carry):
        idx = didx[pl.ds(i * 16, 16)]
        plsc.addupdate_scatter(hist, [idx], ones)
        return carry

    lax.fori_loop(0, _EPW // 16, body, 0)
    pltpu.sync_copy(hist, out_hbm.at[wid])


@functools.partial(
    pl.kernel,
    out_type=jax.ShapeDtypeStruct((_NC, _N, _D), jnp.float32),
    mesh=_sc_mesh,
    scratch_types=[
        pltpu.VMEM((_NCH, _K), jnp.int32),
        pltpu.VMEM((_NCH, _K), jnp.int32),
        pltpu.VMEM((_K, _D), jnp.float32),
        pltpu.VMEM((_ZR, _D), jnp.float32),
        pltpu.VMEM_SHARED((_N, _D), jnp.float32),
        pltpu.SemaphoreType.DMA,
    ],
)
def _edge_kernel(hws_hbm, si2_hbm, di2_hbm, out_hbm, sidx, didx, rows, zbuf,
                 acc, sem):
    cid = lax.axis_index("c")
    sid = lax.axis_index("s")
    wid = sid * _NC + cid

    def zb(i, carry):
        r = i // (_D // 16)
        c = (i % (_D // 16)) * 16
        zbuf[r, pl.ds(c, 16)] = jnp.zeros((16,), jnp.float32)
        return carry

    lax.fori_loop(0, _ZR * (_D // 16), zb, 0)

    def za(j, carry):
        pltpu.sync_copy(zbuf, acc.at[pl.ds(sid * _RPT + j * _ZR, _ZR)])
        return carry

    lax.fori_loop(0, _RPT // _ZR, za, 0)

    pltpu.sync_copy(si2_hbm.at[pl.ds(wid * _NCH, _NCH)], sidx)
    pltpu.sync_copy(di2_hbm.at[pl.ds(wid * _NCH, _NCH)], didx)

    plsc.subcore_barrier()

    def chunk(cn, carry):
        pltpu.async_copy(hws_hbm.at[sidx.at[cn]], rows, sem).wait()
        pltpu.sync_copy(rows, acc.at[didx.at[cn]], add=True)
        return carry

    lax.fori_loop(0, _NCH, chunk, 0)

    plsc.subcore_barrier()
    pltpu.sync_copy(acc.at[pl.ds(sid * _RPT, _RPT)],
                    out_hbm.at[cid, pl.ds(sid * _RPT, _RPT)])


def _dis_body(hist_ref, out_ref):
    deg = jnp.sum(hist_ref[...], axis=0) + 1.0
    out_ref[...] = lax.rsqrt(deg)


_dis_call = pl.pallas_call(
    _dis_body,
    out_shape=jax.ShapeDtypeStruct((_NP // _D, _D), jnp.float32),
)

_RB = 2000


def _pre_body(h_ref, dis_ref, w_ref, g_ref, be_ref, out_ref):
    h = h_ref[...]
    mu = jnp.mean(h, axis=1, keepdims=True)
    var = jnp.mean((h - mu) ** 2, axis=1, keepdims=True)
    hn = (h - mu) / jnp.sqrt(var + 1e-5) * g_ref[...] + be_ref[...]
    ha = jnp.where(hn > 0, hn, 0.2 * hn)
    hw = jnp.dot(ha, w_ref[...], preferred_element_type=jnp.float32)
    out_ref[...] = dis_ref[...] * hw


_pre_call = pl.pallas_call(
    _pre_body,
    grid=(_N // _RB,),
    in_specs=[
        pl.BlockSpec((_RB, _D), lambda i: (i, 0)),
        pl.BlockSpec((_RB, 1), lambda i: (i, 0)),
        pl.BlockSpec((_D, _D), lambda i: (0, 0)),
        pl.BlockSpec((1, _D), lambda i: (0, 0)),
        pl.BlockSpec((1, _D), lambda i: (0, 0)),
    ],
    out_specs=pl.BlockSpec((_RB, _D), lambda i: (i, 0)),
    out_shape=jax.ShapeDtypeStruct((_N, _D), jnp.float32),
)


def _upd_body(acc_ref, hws_ref, dis_ref, h_ref, b_ref, gate_ref, out_ref):
    s = acc_ref[0] + acc_ref[1] + hws_ref[...]
    conv = dis_ref[...] * s + b_ref[...]
    out_ref[...] = h_ref[...] + jax.nn.sigmoid(gate_ref[...]) * conv


def _fin_body(acc_ref, hws_ref, dis_ref, h_ref, b_ref, gate_ref, x0_ref,
              out_ref):
    s = acc_ref[0] + acc_ref[1] + hws_ref[...]
    conv = dis_ref[...] * s + b_ref[...]
    out_ref[...] = (h_ref[...] + jax.nn.sigmoid(gate_ref[...]) * conv
                    + x0_ref[...])


_upd_specs = [
    pl.BlockSpec((2, _RB, _D), lambda i: (0, i, 0)),
    pl.BlockSpec((_RB, _D), lambda i: (i, 0)),
    pl.BlockSpec((_RB, 1), lambda i: (i, 0)),
    pl.BlockSpec((_RB, _D), lambda i: (i, 0)),
    pl.BlockSpec((1, _D), lambda i: (0, 0)),
    pl.BlockSpec((1, 1), lambda i: (0, 0)),
]

_upd_call = pl.pallas_call(
    _upd_body,
    grid=(_N // _RB,),
    in_specs=_upd_specs,
    out_specs=pl.BlockSpec((_RB, _D), lambda i: (i, 0)),
    out_shape=jax.ShapeDtypeStruct((_N, _D), jnp.float32),
)

_fin_call = pl.pallas_call(
    _fin_body,
    grid=(_N // _RB,),
    in_specs=_upd_specs + [pl.BlockSpec((_RB, _D), lambda i: (i, 0))],
    out_specs=pl.BlockSpec((_RB, _D), lambda i: (i, 0)),
    out_shape=jax.ShapeDtypeStruct((_N, _D), jnp.float32),
)


def kernel(x, edge_index, W0, b0, g0, be0, gate0, W1, b1, g1, be1, gate1):
    src = edge_index[0]
    dst = edge_index[1]
    src2 = src.reshape(_NW * _NCH, _K)
    dst2 = dst.reshape(_NW * _NCH, _K)

    hist = _deg_kernel(dst)
    dis80 = _dis_call(hist.reshape(_NW, _NP // _D, _D))
    dis_col = dis80.reshape(_NP)[:_N].reshape(_N, 1)

    h = x
    params = ((W0, b0, g0, be0, gate0), (W1, b1, g1, be1, gate1))
    for li, (W, b, g, be, gate) in enumerate(params):
        hws = _pre_call(h, dis_col, W, g.reshape(1, _D), be.reshape(1, _D))
        acc = _edge_kernel(hws, src2, dst2)
        args = (acc, hws, dis_col, h, b.reshape(1, _D), gate.reshape(1, 1))
        if li == 0:
            h = _upd_call(*args)
        else:
            h = _fin_call(*args, x)
    return h


# speedup vs baseline: 11.1240x; 11.1240x over previous
import functools

import jax
import jax.numpy as jnp
from jax import lax
from jax.experimental import pallas as pl
from jax.experimental.pallas import tpu as pltpu
from jax.experimental.pallas import tpu_sc as plsc

_N = 10000
_E = 320000
_D = 128
_NC = 2
_NS = 16
_NW = _NC * _NS
_EPW = _E // _NW
_K = 80
_NCH = _EPW // _K
_NA = 10240
_RPT = _NA // _NS
_ZR = 32

_sc_mesh = plsc.VectorSubcoreMesh(core_axis_name="c", subcore_axis_name="s")


@functools.partial(
    pl.kernel,
    out_type=jax.ShapeDtypeStruct((_NC, _NA, _D), jnp.float32),
    mesh=_sc_mesh,
    scratch_types=[
        pltpu.VMEM((_K,), jnp.int32),
        pltpu.VMEM((_K, _D), jnp.float32),
        pltpu.VMEM((_ZR, _D), jnp.float32),
        pltpu.VMEM_SHARED((_NA, _D), jnp.float32),
    ],
)
def _deg_kernel(di_hbm, out_hbm, dline, ones_rows, zbuf, acc):
    cid = lax.axis_index("c")
    sid = lax.axis_index("s")
    wid = sid * _NC + cid

    @pl.loop(0, _K * (_D // 16))
    def _fill(i):
        r = i // (_D // 16)
        c = (i % (_D // 16)) * 16
        ones_rows[r, pl.ds(c, 16)] = jnp.ones((16,), jnp.float32)

    @pl.loop(0, _ZR * (_D // 16))
    def _zb(i):
        r = i // (_D // 16)
        c = (i % (_D // 16)) * 16
        zbuf[r, pl.ds(c, 16)] = jnp.zeros((16,), jnp.float32)

    @pl.loop(0, _RPT // _ZR)
    def _za(j):
        pltpu.sync_copy(zbuf, acc.at[pl.ds(sid * _RPT + j * _ZR, _ZR)])

    plsc.subcore_barrier()

    base0 = wid * _EPW

    @pl.loop(0, _NCH)
    def _chunk(cn):
        pltpu.sync_copy(di_hbm.at[pl.ds(base0 + cn * _K, _K)], dline)
        pltpu.sync_copy(ones_rows, acc.at[dline], add=True)

    plsc.subcore_barrier()

    @pl.loop(0, _RPT // _ZR)
    def _out(j):
        r0 = sid * _RPT + j * _ZR
        pltpu.sync_copy(acc.at[pl.ds(r0, _ZR)], zbuf)
        pltpu.sync_copy(zbuf, out_hbm.at[cid, pl.ds(r0, _ZR)])


@functools.partial(
    pl.kernel,
    out_type=jax.ShapeDtypeStruct((_NC, _NA, _D), jnp.float32),
    mesh=_sc_mesh,
    scratch_types=[
        pltpu.VMEM((_K,), jnp.int32),
        pltpu.VMEM((_K,), jnp.int32),
        pltpu.VMEM((_K, _D), jnp.float32),
        pltpu.VMEM((_ZR, _D), jnp.float32),
        pltpu.VMEM_SHARED((_NA, _D), jnp.float32),
        pltpu.SemaphoreType.DMA,
    ],
)
def _edge_kernel(hws_hbm, si_hbm, di_hbm, out_hbm, sline, dline, rows, zbuf,
                 acc, sem):
    cid = lax.axis_index("c")
    sid = lax.axis_index("s")
    wid = sid * _NC + cid

    @pl.loop(0, _ZR * (_D // 16))
    def _zb(i):
        r = i // (_D // 16)
        c = (i % (_D // 16)) * 16
        zbuf[r, pl.ds(c, 16)] = jnp.zeros((16,), jnp.float32)

    @pl.loop(0, _RPT // _ZR)
    def _za(j):
        pltpu.sync_copy(zbuf, acc.at[pl.ds(sid * _RPT + j * _ZR, _ZR)])

    plsc.subcore_barrier()

    base0 = wid * _EPW

    @pl.loop(0, _NCH)
    def _chunk(cn):
        base = base0 + cn * _K
        pltpu.sync_copy(si_hbm.at[pl.ds(base, _K)], sline)
        pltpu.sync_copy(di_hbm.at[pl.ds(base, _K)], dline)
        pltpu.async_copy(hws_hbm.at[sline], rows, sem).wait()
        pltpu.sync_copy(rows, acc.at[dline], add=True)

    plsc.subcore_barrier()

    @pl.loop(0, _RPT // _ZR)
    def _out(j):
        r0 = sid * _RPT + j * _ZR
        pltpu.sync_copy(acc.at[pl.ds(r0, _ZR)], zbuf)
        pltpu.sync_copy(zbuf, out_hbm.at[cid, pl.ds(r0, _ZR)])


_RB = 2000


def _dis_body(parts_ref, out_ref):
    deg = parts_ref[0, :, :1] + parts_ref[1, :, :1] + 1.0
    out_ref[...] = lax.rsqrt(deg)


_dis_call = pl.pallas_call(
    _dis_body,
    grid=(_N // _RB,),
    in_specs=[pl.BlockSpec((2, _RB, _D), lambda i: (0, i, 0))],
    out_specs=pl.BlockSpec((_RB, 1), lambda i: (i, 0)),
    out_shape=jax.ShapeDtypeStruct((_N, 1), jnp.float32),
)


def _pre_body(h_ref, dis_ref, w_ref, g_ref, be_ref, out_ref):
    h = h_ref[...]
    mu = jnp.mean(h, axis=1, keepdims=True)
    var = jnp.mean((h - mu) ** 2, axis=1, keepdims=True)
    hn = (h - mu) / jnp.sqrt(var + 1e-5) * g_ref[...] + be_ref[...]
    ha = jnp.where(hn > 0, hn, 0.2 * hn)
    hw = jnp.dot(ha, w_ref[...], preferred_element_type=jnp.float32)
    out_ref[...] = dis_ref[...] * hw


_pre_call = pl.pallas_call(
    _pre_body,
    grid=(_N // _RB,),
    in_specs=[
        pl.BlockSpec((_RB, _D), lambda i: (i, 0)),
        pl.BlockSpec((_RB, 1), lambda i: (i, 0)),
        pl.BlockSpec((_D, _D), lambda i: (0, 0)),
        pl.BlockSpec((1, _D), lambda i: (0, 0)),
        pl.BlockSpec((1, _D), lambda i: (0, 0)),
    ],
    out_specs=pl.BlockSpec((_RB, _D), lambda i: (i, 0)),
    out_shape=jax.ShapeDtypeStruct((_N, _D), jnp.float32),
)


def _upd_body(acc_ref, hws_ref, dis_ref, h_ref, b_ref, gate_ref, out_ref):
    s = acc_ref[0] + acc_ref[1] + hws_ref[...]
    conv = dis_ref[...] * s + b_ref[...]
    out_ref[...] = h_ref[...] + jax.nn.sigmoid(gate_ref[...]) * conv


def _fin_body(acc_ref, hws_ref, dis_ref, h_ref, b_ref, gate_ref, x0_ref,
              out_ref):
    s = acc_ref[0] + acc_ref[1] + hws_ref[...]
    conv = dis_ref[...] * s + b_ref[...]
    out_ref[...] = (h_ref[...] + jax.nn.sigmoid(gate_ref[...]) * conv
                    + x0_ref[...])


_upd_specs = [
    pl.BlockSpec((2, _RB, _D), lambda i: (0, i, 0)),
    pl.BlockSpec((_RB, _D), lambda i: (i, 0)),
    pl.BlockSpec((_RB, 1), lambda i: (i, 0)),
    pl.BlockSpec((_RB, _D), lambda i: (i, 0)),
    pl.BlockSpec((1, _D), lambda i: (0, 0)),
    pl.BlockSpec((1, 1), lambda i: (0, 0)),
]

_upd_call = pl.pallas_call(
    _upd_body,
    grid=(_N // _RB,),
    in_specs=_upd_specs,
    out_specs=pl.BlockSpec((_RB, _D), lambda i: (i, 0)),
    out_shape=jax.ShapeDtypeStruct((_N, _D), jnp.float32),
)

_fin_call = pl.pallas_call(
    _fin_body,
    grid=(_N // _RB,),
    in_specs=_upd_specs + [pl.BlockSpec((_RB, _D), lambda i: (i, 0))],
    out_specs=pl.BlockSpec((_RB, _D), lambda i: (i, 0)),
    out_shape=jax.ShapeDtypeStruct((_N, _D), jnp.float32),
)


def kernel(x, edge_index, W0, b0, g0, be0, gate0, W1, b1, g1, be1, gate1):
    src = edge_index[0]
    dst = edge_index[1]

    deg_parts = _deg_kernel(dst)
    dis_col = _dis_call(deg_parts)

    h = x
    params = ((W0, b0, g0, be0, gate0), (W1, b1, g1, be1, gate1))
    for li, (W, b, g, be, gate) in enumerate(params):
        hws = _pre_call(h, dis_col, W, g.reshape(1, _D), be.reshape(1, _D))
        acc = _edge_kernel(hws, src, dst)
        args = (acc, hws, dis_col, h, b.reshape(1, _D), gate.reshape(1, 1))
        if li == 0:
            h = _upd_call(*args)
        else:
            h = _fin_call(*args, x)
    return h


# speedup vs baseline: 23.3735x; 2.1012x over previous
import functools

import jax
import jax.numpy as jnp
from jax import lax
from jax.experimental import pallas as pl
from jax.experimental.pallas import tpu as pltpu
from jax.experimental.pallas import tpu_sc as plsc

_N = 10000
_E = 320000
_D = 128
_NC = 2
_NS = 16
_NW = _NC * _NS
_EPW = _E // _NW
_K = 80
_NCH = _EPW // _K
_NA = 10240
_RPT = _NA // _NS
_ZR = 32

_sc_mesh = plsc.VectorSubcoreMesh(core_axis_name="c", subcore_axis_name="s")


@functools.partial(
    pl.kernel,
    out_type=jax.ShapeDtypeStruct((_NC, _NA, _D), jnp.float32),
    mesh=_sc_mesh,
    scratch_types=[
        pltpu.VMEM((_NCH, _K), jnp.int32),
        pltpu.VMEM((_K, _D), jnp.float32),
        pltpu.VMEM((8, _D), jnp.float32),
        pltpu.VMEM_SHARED((_NA, _D), jnp.float32),
    ],
)
def _deg_kernel(di3_hbm, out_hbm, didx, ones_rows, zbuf, acc):
    cid = lax.axis_index("c")
    sid = lax.axis_index("s")
    wid = sid * _NC + cid

    @pl.loop(0, _K * (_D // 16))
    def _fo(i):
        r = i // (_D // 16)
        c = (i % (_D // 16)) * 16
        ones_rows[r, pl.ds(c, 16)] = jnp.ones((16,), jnp.float32)

    @pl.loop(0, 8 * (_D // 16))
    def _fz(i):
        r = i // (_D // 16)
        c = (i % (_D // 16)) * 16
        zbuf[r, pl.ds(c, 16)] = jnp.zeros((16,), jnp.float32)

    @pl.loop(0, _RPT // 8)
    def _za(j):
        pltpu.sync_copy(zbuf, acc.at[pl.ds(sid * _RPT + j * 8, 8)])

    pltpu.sync_copy(di3_hbm.at[wid], didx)

    plsc.subcore_barrier()

    @pl.loop(0, _NCH)
    def _chunk(cn):
        pltpu.sync_copy(ones_rows, acc.at[didx.at[cn]], add=True)

    plsc.subcore_barrier()
    r0 = sid * _RPT
    pltpu.sync_copy(acc.at[pl.ds(r0, _RPT)], out_hbm.at[cid, pl.ds(r0, _RPT)])


@functools.partial(
    pl.kernel,
    out_type=jax.ShapeDtypeStruct((_NC, _NA, _D), jnp.float32),
    mesh=_sc_mesh,
    scratch_types=[
        pltpu.VMEM((_EPW,), jnp.int32),
        pltpu.VMEM((_NCH, _K), jnp.int32),
        pltpu.VMEM((_K, _D), jnp.float32),
        pltpu.VMEM((_K, _D), jnp.float32),
        pltpu.VMEM_SHARED((_NA, _D), jnp.float32),
        pltpu.SemaphoreType.DMA,
        pltpu.SemaphoreType.DMA,
    ],
)
def _edge_kernel(hws_hbm, si_hbm, di3_hbm, out_hbm, sidx, didx, rows0, rows1,
                 acc, sem0, sem1):
    cid = lax.axis_index("c")
    sid = lax.axis_index("s")
    wid = sid * _NC + cid

    @pl.loop(0, _K * (_D // 16))
    def _zr(i):
        r = i // (_D // 16)
        c = (i % (_D // 16)) * 16
        rows0[r, pl.ds(c, 16)] = jnp.zeros((16,), jnp.float32)

    @pl.loop(0, _RPT // _K)
    def _za(j):
        pltpu.sync_copy(rows0, acc.at[pl.ds(sid * _RPT + j * _K, _K)])

    pltpu.sync_copy(si_hbm.at[pl.ds(wid * _EPW, _EPW)], sidx)
    pltpu.sync_copy(di3_hbm.at[wid], didx)

    plsc.subcore_barrier()

    pltpu.async_copy(hws_hbm.at[sidx.at[pl.ds(0, _K)]], rows0, sem0)
    pltpu.async_copy(hws_hbm.at[sidx.at[pl.ds(_K, _K)]], rows1, sem1)

    @pl.loop(0, _NCH - 1, step=2)
    def _chunk(cn):
        for b, (rows, sem) in enumerate(((rows0, sem0), (rows1, sem1))):
            c = cn + b
            pltpu.make_async_copy(
                hws_hbm.at[sidx.at[pl.ds(c * _K, _K)]], rows, sem).wait()
            pltpu.sync_copy(rows, acc.at[didx.at[c]], add=True)

            @pl.when(c + 2 < _NCH)
            def _():
                pltpu.async_copy(
                    hws_hbm.at[sidx.at[pl.ds((c + 2) * _K, _K)]], rows, sem)

    pltpu.make_async_copy(
        hws_hbm.at[sidx.at[pl.ds((_NCH - 1) * _K, _K)]], rows0, sem0).wait()
    pltpu.sync_copy(rows0, acc.at[didx.at[_NCH - 1]], add=True)

    plsc.subcore_barrier()
    r0 = sid * _RPT
    pltpu.sync_copy(acc.at[pl.ds(r0, _RPT)], out_hbm.at[cid, pl.ds(r0, _RPT)])


_RB = 2000


def _dis_body(parts_ref, out_ref):
    deg = parts_ref[0, :, :1] + parts_ref[1, :, :1] + 1.0
    out_ref[...] = lax.rsqrt(deg)


_dis_call = pl.pallas_call(
    _dis_body,
    grid=(_N // _RB,),
    in_specs=[pl.BlockSpec((2, _RB, _D), lambda i: (0, i, 0))],
    out_specs=pl.BlockSpec((_RB, 1), lambda i: (i, 0)),
    out_shape=jax.ShapeDtypeStruct((_N, 1), jnp.float32),
)


def _pre_body(h_ref, dis_ref, w_ref, g_ref, be_ref, out_ref):
    h = h_ref[...]
    mu = jnp.mean(h, axis=1, keepdims=True)
    var = jnp.mean((h - mu) ** 2, axis=1, keepdims=True)
    hn = (h - mu) / jnp.sqrt(var + 1e-5) * g_ref[...] + be_ref[...]
    ha = jnp.where(hn > 0, hn, 0.2 * hn)
    hw = jnp.dot(ha, w_ref[...], preferred_element_type=jnp.float32)
    out_ref[...] = dis_ref[...] * hw


_pre_call = pl.pallas_call(
    _pre_body,
    grid=(_N // _RB,),
    in_specs=[
        pl.BlockSpec((_RB, _D), lambda i: (i, 0)),
        pl.BlockSpec((_RB, 1), lambda i: (i, 0)),
        pl.BlockSpec((_D, _D), lambda i: (0, 0)),
        pl.BlockSpec((1, _D), lambda i: (0, 0)),
        pl.BlockSpec((1, _D), lambda i: (0, 0)),
    ],
    out_specs=pl.BlockSpec((_RB, _D), lambda i: (i, 0)),
    out_shape=jax.ShapeDtypeStruct((_N, _D), jnp.float32),
)


def _upd_body(acc_ref, hws_ref, dis_ref, h_ref, b_ref, gate_ref, out_ref):
    s = acc_ref[0] + acc_ref[1] + hws_ref[...]
    conv = dis_ref[...] * s + b_ref[...]
    out_ref[...] = h_ref[...] + jax.nn.sigmoid(gate_ref[...]) * conv


def _fin_body(acc_ref, hws_ref, dis_ref, h_ref, b_ref, gate_ref, x0_ref,
              out_ref):
    s = acc_ref[0] + acc_ref[1] + hws_ref[...]
    conv = dis_ref[...] * s + b_ref[...]
    out_ref[...] = (h_ref[...] + jax.nn.sigmoid(gate_ref[...]) * conv
                    + x0_ref[...])


_upd_specs = [
    pl.BlockSpec((2, _RB, _D), lambda i: (0, i, 0)),
    pl.BlockSpec((_RB, _D), lambda i: (i, 0)),
    pl.BlockSpec((_RB, 1), lambda i: (i, 0)),
    pl.BlockSpec((_RB, _D), lambda i: (i, 0)),
    pl.BlockSpec((1, _D), lambda i: (0, 0)),
    pl.BlockSpec((1, 1), lambda i: (0, 0)),
]

_upd_call = pl.pallas_call(
    _upd_body,
    grid=(_N // _RB,),
    in_specs=_upd_specs,
    out_specs=pl.BlockSpec((_RB, _D), lambda i: (i, 0)),
    out_shape=jax.ShapeDtypeStruct((_N, _D), jnp.float32),
)

_fin_call = pl.pallas_call(
    _fin_body,
    grid=(_N // _RB,),
    in_specs=_upd_specs + [pl.BlockSpec((_RB, _D), lambda i: (i, 0))],
    out_specs=pl.BlockSpec((_RB, _D), lambda i: (i, 0)),
    out_shape=jax.ShapeDtypeStruct((_N, _D), jnp.float32),
)


def kernel(x, edge_index, W0, b0, g0, be0, gate0, W1, b1, g1, be1, gate1):
    src = edge_index[0]
    dst = edge_index[1]
    dst3 = dst.reshape(_NW, _NCH, _K)

    deg_parts = _deg_kernel(dst3)
    dis_col = _dis_call(deg_parts)

    h = x
    params = ((W0, b0, g0, be0, gate0), (W1, b1, g1, be1, gate1))
    for li, (W, b, g, be, gate) in enumerate(params):
        hws = _pre_call(h, dis_col, W, g.reshape(1, _D), be.reshape(1, _D))
        acc = _edge_kernel(hws, src, dst3)
        args = (acc, hws, dis_col, h, b.reshape(1, _D), gate.reshape(1, 1))
        if li == 0:
            h = _upd_call(*args)
        else:
            h = _fin_call(*args, x)
    return h


# speedup vs baseline: 26.5163x; 1.1345x over previous
import functools

import jax
import jax.numpy as jnp
from jax import lax
from jax.experimental import pallas as pl
from jax.experimental.pallas import tpu as pltpu
from jax.experimental.pallas import tpu_sc as plsc

_N = 10000
_E = 320000
_D = 128
_NC = 2
_NS = 16
_NW = _NC * _NS
_EPW = _E // _NW
_K = 80
_NCH = _EPW // _K
_NA = 10240
_RPT = _NA // _NS
_ZR = 32

_sc_mesh = plsc.VectorSubcoreMesh(core_axis_name="c", subcore_axis_name="s")


@functools.partial(
    pl.kernel,
    out_type=jax.ShapeDtypeStruct((_NC, _NA, _D), jnp.float32),
    mesh=_sc_mesh,
    scratch_types=[
        pltpu.VMEM((_NCH, _K), jnp.int32),
        pltpu.VMEM((_K, _D), jnp.float32),
        pltpu.VMEM((8, _D), jnp.float32),
        pltpu.VMEM_SHARED((_NA, _D), jnp.float32),
    ],
)
def _deg_kernel(di3_hbm, out_hbm, didx, ones_rows, zbuf, acc):
    cid = lax.axis_index("c")
    sid = lax.axis_index("s")
    wid = sid * _NC + cid

    @pl.loop(0, _K * (_D // 16))
    def _fo(i):
        r = i // (_D // 16)
        c = (i % (_D // 16)) * 16
        ones_rows[r, pl.ds(c, 16)] = jnp.ones((16,), jnp.float32)

    @pl.loop(0, 8 * (_D // 16))
    def _fz(i):
        r = i // (_D // 16)
        c = (i % (_D // 16)) * 16
        zbuf[r, pl.ds(c, 16)] = jnp.zeros((16,), jnp.float32)

    @pl.loop(0, _RPT // 8)
    def _za(j):
        pltpu.sync_copy(zbuf, acc.at[pl.ds(sid * _RPT + j * 8, 8)])

    pltpu.sync_copy(di3_hbm.at[wid], didx)

    plsc.subcore_barrier()

    @pl.loop(0, _NCH)
    def _chunk(cn):
        pltpu.sync_copy(ones_rows, acc.at[didx.at[cn]], add=True)

    plsc.subcore_barrier()
    r0 = sid * _RPT
    pltpu.sync_copy(acc.at[pl.ds(r0, _RPT)], out_hbm.at[cid, pl.ds(r0, _RPT)])


@functools.partial(
    pl.kernel,
    out_type=jax.ShapeDtypeStruct((_NC, _NA, _D), jnp.float32),
    mesh=_sc_mesh,
    scratch_types=[
        pltpu.VMEM((_EPW,), jnp.int32),
        pltpu.VMEM((_K,), jnp.int32),
        pltpu.VMEM((_K,), jnp.int32),
        pltpu.VMEM((_K,), jnp.int32),
        pltpu.VMEM((_K, _D), jnp.float32),
        pltpu.VMEM((_K, _D), jnp.float32),
        pltpu.VMEM((_K, _D), jnp.float32),
        pltpu.VMEM_SHARED((_NA, _D), jnp.float32),
        pltpu.SemaphoreType.DMA,
        pltpu.SemaphoreType.DMA,
        pltpu.SemaphoreType.DMA,
        pltpu.SemaphoreType.DMA,
        pltpu.SemaphoreType.DMA,
        pltpu.SemaphoreType.DMA,
    ],
)
def _edge_kernel(hws_hbm, si_hbm, di_hbm, out_hbm, sidx, dl0, dl1, dl2,
                 rows0, rows1, rows2, acc, sg0, sg1, sg2, si0, si1, si2):
    cid = lax.axis_index("c")
    sid = lax.axis_index("s")
    wid = sid * _NC + cid

    dls = (dl0, dl1, dl2)
    rws = (rows0, rows1, rows2)
    sgs = (sg0, sg1, sg2)
    sis = (si0, si1, si2)

    @pl.loop(0, _K * (_D // 16))
    def _zr(i):
        r = i // (_D // 16)
        c = (i % (_D // 16)) * 16
        rows0[r, pl.ds(c, 16)] = jnp.zeros((16,), jnp.float32)

    @pl.loop(0, _RPT // _K)
    def _za(j):
        pltpu.sync_copy(rows0, acc.at[pl.ds(sid * _RPT + j * _K, _K)])

    base0 = wid * _EPW
    pltpu.sync_copy(si_hbm.at[pl.ds(base0, _EPW)], sidx)

    plsc.subcore_barrier()

    def issue(c, b):
        pltpu.async_copy(di_hbm.at[pl.ds(base0 + c * _K, _K)], dls[b], sis[b])
        pltpu.async_copy(hws_hbm.at[sidx.at[pl.ds(c * _K, _K)]], rws[b],
                         sgs[b])

    def consume(c, b):
        pltpu.make_async_copy(di_hbm.at[pl.ds(base0 + c * _K, _K)], dls[b],
                              sis[b]).wait()
        pltpu.make_async_copy(hws_hbm.at[sidx.at[pl.ds(c * _K, _K)]], rws[b],
                              sgs[b]).wait()
        pltpu.sync_copy(rws[b], acc.at[dls[b]], add=True)

    for b in range(3):
        issue(b, b)

    @pl.loop(0, _NCH - 2, step=3)
    def _chunk(cn):
        for b in range(3):
            c = cn + b
            consume(c, b)

            @pl.when(c + 3 < _NCH)
            def _():
                issue(c + 3, b)

    consume(_NCH - 2, 0)
    consume(_NCH - 1, 1)

    plsc.subcore_barrier()
    r0 = sid * _RPT
    pltpu.sync_copy(acc.at[pl.ds(r0, _RPT)], out_hbm.at[cid, pl.ds(r0, _RPT)])


_RB = 2000


def _dis_body(parts_ref, out_ref):
    deg = parts_ref[0, :, :1] + parts_ref[1, :, :1] + 1.0
    out_ref[...] = lax.rsqrt(deg)


_dis_call = pl.pallas_call(
    _dis_body,
    grid=(_N // _RB,),
    in_specs=[pl.BlockSpec((2, _RB, _D), lambda i: (0, i, 0))],
    out_specs=pl.BlockSpec((_RB, 1), lambda i: (i, 0)),
    out_shape=jax.ShapeDtypeStruct((_N, 1), jnp.float32),
)


def _pre_body(h_ref, dis_ref, w_ref, g_ref, be_ref, out_ref):
    h = h_ref[...]
    mu = jnp.mean(h, axis=1, keepdims=True)
    var = jnp.mean((h - mu) ** 2, axis=1, keepdims=True)
    hn = (h - mu) / jnp.sqrt(var + 1e-5) * g_ref[...] + be_ref[...]
    ha = jnp.where(hn > 0, hn, 0.2 * hn)
    hw = jnp.dot(ha, w_ref[...], preferred_element_type=jnp.float32)
    out_ref[...] = dis_ref[...] * hw


_pre_call = pl.pallas_call(
    _pre_body,
    grid=(_N // _RB,),
    in_specs=[
        pl.BlockSpec((_RB, _D), lambda i: (i, 0)),
        pl.BlockSpec((_RB, 1), lambda i: (i, 0)),
        pl.BlockSpec((_D, _D), lambda i: (0, 0)),
        pl.BlockSpec((1, _D), lambda i: (0, 0)),
        pl.BlockSpec((1, _D), lambda i: (0, 0)),
    ],
    out_specs=pl.BlockSpec((_RB, _D), lambda i: (i, 0)),
    out_shape=jax.ShapeDtypeStruct((_N, _D), jnp.float32),
)


def _upd_body(acc_ref, hws_ref, dis_ref, h_ref, b_ref, gate_ref, out_ref):
    s = acc_ref[0] + acc_ref[1] + hws_ref[...]
    conv = dis_ref[...] * s + b_ref[...]
    out_ref[...] = h_ref[...] + jax.nn.sigmoid(gate_ref[...]) * conv


def _fin_body(acc_ref, hws_ref, dis_ref, h_ref, b_ref, gate_ref, x0_ref,
              out_ref):
    s = acc_ref[0] + acc_ref[1] + hws_ref[...]
    conv = dis_ref[...] * s + b_ref[...]
    out_ref[...] = (h_ref[...] + jax.nn.sigmoid(gate_ref[...]) * conv
                    + x0_ref[...])


_upd_specs = [
    pl.BlockSpec((2, _RB, _D), lambda i: (0, i, 0)),
    pl.BlockSpec((_RB, _D), lambda i: (i, 0)),
    pl.BlockSpec((_RB, 1), lambda i: (i, 0)),
    pl.BlockSpec((_RB, _D), lambda i: (i, 0)),
    pl.BlockSpec((1, _D), lambda i: (0, 0)),
    pl.BlockSpec((1, 1), lambda i: (0, 0)),
]

_upd_call = pl.pallas_call(
    _upd_body,
    grid=(_N // _RB,),
    in_specs=_upd_specs,
    out_specs=pl.BlockSpec((_RB, _D), lambda i: (i, 0)),
    out_shape=jax.ShapeDtypeStruct((_N, _D), jnp.float32),
)

_fin_call = pl.pallas_call(
    _fin_body,
    grid=(_N // _RB,),
    in_specs=_upd_specs + [pl.BlockSpec((_RB, _D), lambda i: (i, 0))],
    out_specs=pl.BlockSpec((_RB, _D), lambda i: (i, 0)),
    out_shape=jax.ShapeDtypeStruct((_N, _D), jnp.float32),
)


def kernel(x, edge_index, W0, b0, g0, be0, gate0, W1, b1, g1, be1, gate1):
    src = edge_index[0]
    dst = edge_index[1]
    dst3 = dst.reshape(_NW, _NCH, _K)

    deg_parts = _deg_kernel(dst3)
    dis_col = _dis_call(deg_parts)

    h = x
    params = ((W0, b0, g0, be0, gate0), (W1, b1, g1, be1, gate1))
    for li, (W, b, g, be, gate) in enumerate(params):
        hws = _pre_call(h, dis_col, W, g.reshape(1, _D), be.reshape(1, _D))
        acc = _edge_kernel(hws, src, dst)
        args = (acc, hws, dis_col, h, b.reshape(1, _D), gate.reshape(1, 1))
        if li == 0:
            h = _upd_call(*args)
        else:
            h = _fin_call(*args, x)
    return h


# speedup vs baseline: 27.3188x; 1.0303x over previous
import functools

import jax
import jax.numpy as jnp
from jax import lax
from jax.experimental import pallas as pl
from jax.experimental.pallas import tpu as pltpu
from jax.experimental.pallas import tpu_sc as plsc

_N = 10000
_E = 320000
_D = 128
_NC = 2
_NS = 16
_NW = _NC * _NS
_EPW = _E // _NW
_K = 80
_NCH = _EPW // _K
_NA = 10240
_RPT = _NA // _NS
_ZR = 32

_sc_mesh = plsc.VectorSubcoreMesh(core_axis_name="c", subcore_axis_name="s")


@functools.partial(
    pl.kernel,
    out_type=jax.ShapeDtypeStruct((_NC, _NA, _D), jnp.float32),
    mesh=_sc_mesh,
    scratch_types=[
        pltpu.VMEM((_NCH, _K), jnp.int32),
        pltpu.VMEM((_K, _D), jnp.float32),
        pltpu.VMEM((8, _D), jnp.float32),
        pltpu.VMEM_SHARED((_NA, _D), jnp.float32),
    ],
)
def _deg_kernel(di3_hbm, out_hbm, didx, ones_rows, zbuf, acc):
    cid = lax.axis_index("c")
    sid = lax.axis_index("s")
    wid = sid * _NC + cid

    @pl.loop(0, _K * (_D // 16))
    def _fo(i):
        r = i // (_D // 16)
        c = (i % (_D // 16)) * 16
        ones_rows[r, pl.ds(c, 16)] = jnp.ones((16,), jnp.float32)

    @pl.loop(0, 8 * (_D // 16))
    def _fz(i):
        r = i // (_D // 16)
        c = (i % (_D // 16)) * 16
        zbuf[r, pl.ds(c, 16)] = jnp.zeros((16,), jnp.float32)

    @pl.loop(0, _RPT // 8)
    def _za(j):
        pltpu.sync_copy(zbuf, acc.at[pl.ds(sid * _RPT + j * 8, 8)])

    pltpu.sync_copy(di3_hbm.at[wid], didx)

    plsc.subcore_barrier()

    @pl.loop(0, _NCH)
    def _chunk(cn):
        pltpu.sync_copy(ones_rows, acc.at[didx.at[cn]], add=True)

    plsc.subcore_barrier()
    r0 = sid * _RPT
    pltpu.sync_copy(acc.at[pl.ds(r0, _RPT)], out_hbm.at[cid, pl.ds(r0, _RPT)])


@functools.partial(
    pl.kernel,
    out_type=jax.ShapeDtypeStruct((_NC, _NA, _D), jnp.float32),
    mesh=_sc_mesh,
    scratch_types=[
        pltpu.VMEM((_EPW,), jnp.int32),
        pltpu.VMEM((_K,), jnp.int32),
        pltpu.VMEM((_K,), jnp.int32),
        pltpu.VMEM((_K,), jnp.int32),
        pltpu.VMEM((_K, _D), jnp.float32),
        pltpu.VMEM((_K, _D), jnp.float32),
        pltpu.VMEM((_K, _D), jnp.float32),
        pltpu.VMEM_SHARED((_NA, _D), jnp.float32),
        pltpu.SemaphoreType.DMA,
        pltpu.SemaphoreType.DMA,
        pltpu.SemaphoreType.DMA,
        pltpu.SemaphoreType.DMA,
        pltpu.SemaphoreType.DMA,
        pltpu.SemaphoreType.DMA,
    ],
)
def _edge_kernel(hws_hbm, si_hbm, di_hbm, out_hbm, sidx, dl0, dl1, dl2,
                 rows0, rows1, rows2, acc, sg0, sg1, sg2, si0, si1, si2):
    cid = lax.axis_index("c")
    sid = lax.axis_index("s")
    wid = sid * _NC + cid

    dls = (dl0, dl1, dl2)
    rws = (rows0, rows1, rows2)
    sgs = (sg0, sg1, sg2)
    sis = (si0, si1, si2)

    @pl.loop(0, _K * (_D // 16))
    def _zr(i):
        r = i // (_D // 16)
        c = (i % (_D // 16)) * 16
        rows0[r, pl.ds(c, 16)] = jnp.zeros((16,), jnp.float32)

    @pl.loop(0, _RPT // _K)
    def _za(j):
        pltpu.sync_copy(rows0, acc.at[pl.ds(sid * _RPT + j * _K, _K)])

    base0 = wid * _EPW
    pltpu.sync_copy(si_hbm.at[pl.ds(base0, _EPW)], sidx)

    plsc.subcore_barrier()

    def issue(c, b):
        pltpu.async_copy(di_hbm.at[pl.ds(base0 + c * _K, _K)], dls[b], sis[b])
        pltpu.async_copy(hws_hbm.at[sidx.at[pl.ds(c * _K, _K)]], rws[b],
                         sgs[b])

    def consume(c, b):
        pltpu.make_async_copy(di_hbm.at[pl.ds(base0 + c * _K, _K)], dls[b],
                              sis[b]).wait()
        pltpu.make_async_copy(hws_hbm.at[sidx.at[pl.ds(c * _K, _K)]], rws[b],
                              sgs[b]).wait()
        pltpu.sync_copy(rws[b], acc.at[dls[b]], add=True)

    for b in range(3):
        issue(b, b)

    @pl.loop(0, _NCH - 2, step=3)
    def _chunk(cn):
        for b in range(3):
            c = cn + b
            consume(c, b)

            @pl.when(c + 3 < _NCH)
            def _():
                issue(c + 3, b)

    consume(_NCH - 2, 0)
    consume(_NCH - 1, 1)

    plsc.subcore_barrier()
    r0 = sid * _RPT
    pltpu.sync_copy(acc.at[pl.ds(r0, _RPT)], out_hbm.at[cid, pl.ds(r0, _RPT)])


_RB = 2000

_rowspec = pl.BlockSpec((_RB, _D), lambda i: (i, 0))
_colspec = pl.BlockSpec((_RB, 1), lambda i: (i, 0))
_accspec = pl.BlockSpec((2, _RB, _D), lambda i: (0, i, 0))
_wspec = pl.BlockSpec((_D, _D), lambda i: (0, 0))
_vspec = pl.BlockSpec((1, _D), lambda i: (0, 0))
_sspec = pl.BlockSpec((1, 1), lambda i: (0, 0))


def _dense(h, dis, w, g, be):
    mu = jnp.mean(h, axis=1, keepdims=True)
    var = jnp.mean((h - mu) ** 2, axis=1, keepdims=True)
    hn = (h - mu) / jnp.sqrt(var + 1e-5) * g + be
    ha = jnp.where(hn > 0, hn, 0.2 * hn)
    hw = jnp.dot(ha, w, preferred_element_type=jnp.float32)
    return dis * hw


def _pre1_body(parts_ref, h_ref, w_ref, g_ref, be_ref, hws_ref, dis_ref):
    deg = parts_ref[0, :, :1] + parts_ref[1, :, :1] + 1.0
    dis = lax.rsqrt(deg)
    dis_ref[...] = dis
    hws_ref[...] = _dense(h_ref[...], dis, w_ref[...], g_ref[...], be_ref[...])


_pre1_call = pl.pallas_call(
    _pre1_body,
    grid=(_N // _RB,),
    in_specs=[_accspec, _rowspec, _wspec, _vspec, _vspec],
    out_specs=(_rowspec, _colspec),
    out_shape=(jax.ShapeDtypeStruct((_N, _D), jnp.float32),
               jax.ShapeDtypeStruct((_N, 1), jnp.float32)),
)


def _mid_body(acc_ref, hws_ref, dis_ref, h_ref, b_ref, gate_ref, w_ref,
              g_ref, be_ref, h1_ref, hws2_ref):
    ssum = acc_ref[0] + acc_ref[1] + hws_ref[...]
    conv = dis_ref[...] * ssum + b_ref[...]
    h1 = h_ref[...] + jax.nn.sigmoid(gate_ref[...]) * conv
    h1_ref[...] = h1
    hws2_ref[...] = _dense(h1, dis_ref[...], w_ref[...], g_ref[...],
                           be_ref[...])


_mid_call = pl.pallas_call(
    _mid_body,
    grid=(_N // _RB,),
    in_specs=[_accspec, _rowspec, _colspec, _rowspec, _vspec, _sspec,
              _wspec, _vspec, _vspec],
    out_specs=(_rowspec, _rowspec),
    out_shape=(jax.ShapeDtypeStruct((_N, _D), jnp.float32),
               jax.ShapeDtypeStruct((_N, _D), jnp.float32)),
)


def _fin_body(acc_ref, hws_ref, dis_ref, h_ref, b_ref, gate_ref, x0_ref,
              out_ref):
    ssum = acc_ref[0] + acc_ref[1] + hws_ref[...]
    conv = dis_ref[...] * ssum + b_ref[...]
    out_ref[...] = (h_ref[...] + jax.nn.sigmoid(gate_ref[...]) * conv
                    + x0_ref[...])


_fin_call = pl.pallas_call(
    _fin_body,
    grid=(_N // _RB,),
    in_specs=[_accspec, _rowspec, _colspec, _rowspec, _vspec, _sspec,
              _rowspec],
    out_specs=_rowspec,
    out_shape=jax.ShapeDtypeStruct((_N, _D), jnp.float32),
)


def kernel(x, edge_index, W0, b0, g0, be0, gate0, W1, b1, g1, be1, gate1):
    src = edge_index[0]
    dst = edge_index[1]
    dst3 = dst.reshape(_NW, _NCH, _K)

    deg_parts = _deg_kernel(dst3)
    hws1, dis_col = _pre1_call(deg_parts, x, W0, g0.reshape(1, _D),
                               be0.reshape(1, _D))
    acc1 = _edge_kernel(hws1, src, dst)
    h1, hws2 = _mid_call(acc1, hws1, dis_col, x, b0.reshape(1, _D),
                         gate0.reshape(1, 1), W1, g1.reshape(1, _D),
                         be1.reshape(1, _D))
    acc2 = _edge_kernel(hws2, src, dst)
    return _fin_call(acc2, hws2, dis_col, h1, b1.reshape(1, _D),
                     gate1.reshape(1, 1), x)
